# baseline (speedup 1.0000x reference)
import functools

import jax
import jax.numpy as jnp
from jax import lax
from jax.experimental import pallas as pl
from jax.experimental.pallas import tpu as pltpu
from jax.experimental.pallas import tpu_sc as plsc

TC_FRAC_NUM, TC_FRAC_DEN = 1, 2


@functools.cache
def _build_sc(B, P, F, D, S):
    info = plsc.get_sparse_core_info()
    NC, NS, L = info.num_cores, info.num_subcores, info.num_lanes
    NW = NC * NS
    R = B * F - S
    mrows = (B * F) // NW
    grows = R // NW
    CHUNK = 64
    NBUF = 3
    nchunk = grows // CHUNK
    mesh = plsc.VectorSubcoreMesh(core_axis_name="c", subcore_axis_name="s")

    @functools.partial(
        pl.kernel,
        mesh=mesh,
        out_type=[
            jax.ShapeDtypeStruct((R, D), jnp.float32),
            jax.ShapeDtypeStruct((B * F,), jnp.int32),
        ],
        scratch_types=[
            pltpu.VMEM((mrows,), jnp.int32),
            pltpu.VMEM((mrows,), jnp.int32),
            pltpu.VMEM((max(grows, L),), jnp.int32),
            pltpu.VMEM((NBUF, CHUNK, D), jnp.float32),
            pltpu.SemaphoreType.DMA,
            pltpu.SemaphoreType.DMA,
        ],
    )
    def lr(x_hbm, vi_hbm, out_hbm, msk_hbm, mvi_v, msk_v, idx_v, rows_v,
           gsem, ssem):
        wid = lax.axis_index("s") * NC + lax.axis_index("c")
        gbase = S + wid * grows
        obase = wid * grows
        if grows:
            pltpu.sync_copy(vi_hbm.at[pl.ds(gbase, grows)], idx_v)

        def build(g):
            for j in range(CHUNK // L):
                i = g * (CHUNK // L) + j
                v = idx_v[pl.ds(i * L, L)]
                boff = ((gbase + i * L) // F) * P
                idx_v[pl.ds(i * L, L)] = v + boff

        gathers, scatters = [], []

        def start_gather(g):
            gathers.append(pltpu.async_copy(
                x_hbm.at[idx_v.at[pl.ds(g * CHUNK, CHUNK)]],
                rows_v.at[g % NBUF], gsem))

        def start_scatter(g):
            gathers[g].wait()
            scatters.append(pltpu.async_copy(
                rows_v.at[g % NBUF],
                out_hbm.at[pl.ds(obase + g * CHUNK, CHUNK)], ssem))

        for g in range(nchunk):
            build(g)
            if g >= NBUF:
                scatters[g - NBUF].wait()
            start_gather(g)
            if g >= 1:
                start_scatter(g - 1)

        mbase = wid * mrows
        pltpu.sync_copy(vi_hbm.at[pl.ds(mbase, mrows)], mvi_v)
        for i in range(mrows // L):
            v = mvi_v[pl.ds(i * L, L)]
            msk_v[pl.ds(i * L, L)] = jnp.where(
                v != P - 1,
                jnp.full((L,), 1, jnp.int32),
                jnp.full((L,), 0, jnp.int32),
            )
        msk_copy = pltpu.async_copy(msk_v, msk_hbm.at[pl.ds(mbase, mrows)],
                                    gsem)
        if nchunk:
            start_scatter(nchunk - 1)
        for g in range(max(0, nchunk - NBUF), nchunk):
            scatters[g].wait()
        msk_copy.wait()

    return lr


@functools.cache
def _build_tc(B, P, F, D, S):
    CF = 128
    bpf = F // CF
    nblk = S // CF

    def tc(vi_ref, x_any, out_ref, sem):
        i = pl.program_id(0)
        boff = (i // bpf) * P
        copies = []
        for r in range(CF):
            row = vi_ref[i * CF + r] + boff
            cp = pltpu.make_async_copy(
                x_any.at[pl.ds(row, 1), :],
                out_ref.at[pl.ds(r, 1), :], sem)
            cp.start()
            copies.append(cp)
        for cp in copies:
            cp.wait()

    grid_spec = pltpu.PrefetchScalarGridSpec(
        num_scalar_prefetch=1,
        grid=(nblk,),
        in_specs=[pl.BlockSpec(memory_space=pl.ANY)],
        out_specs=pl.BlockSpec((CF, D), lambda i, vi: (i, 0)),
        scratch_shapes=[pltpu.SemaphoreType.DMA],
    )
    return pl.pallas_call(
        tc,
        grid_spec=grid_spec,
        out_shape=jax.ShapeDtypeStruct((S, D), jnp.float32),
    )


def kernel(x, durations, val_ind):
    del durations
    B, P, D = x.shape
    F = val_ind.shape[1]
    S = (B * F * TC_FRAC_NUM // TC_FRAC_DEN) // 128 * 128
    x_flat = x.reshape(B * P, D)
    vi_flat = val_ind.reshape(B * F)
    sc_out, msk = _build_sc(B, P, F, D, S)(x_flat, vi_flat)
    tc_out = _build_tc(B, P, F, D, S)(vi_flat[:S], x_flat)
    out_flat = jnp.concatenate([tc_out, sc_out], axis=0)
    return out_flat.reshape(B, F, D), msk.astype(bool).reshape(B, F, 1)
